# Initial kernel scaffold; baseline (speedup 1.0000x reference)
import functools

import jax
import jax.numpy as jnp
from jax import lax
from jax.experimental import pallas as pl
from jax.experimental.pallas import tpu as pltpu
from jax.experimental.pallas import tpu_sc as plsc

NC = 2
NS = 16
NW = NC * NS
LANE = 16
ROW = 128


def _cdiv(a, b):
  return (a + b - 1) // b


def _make_deg_kernel(pad_n, rows_per_w):
  rows_per_tile = pad_n // NS
  zchunk = 64
  mesh = plsc.VectorSubcoreMesh(core_axis_name="c", subcore_axis_name="s")

  @functools.partial(
      pl.kernel,
      out_type=jax.ShapeDtypeStruct((NC, pad_n, LANE), jnp.float32),
      mesh=mesh,
      scratch_types=[
          pltpu.VMEM((rows_per_w, ROW), jnp.int32),
          pltpu.VMEM((ROW, LANE), jnp.float32),
          pltpu.VMEM((zchunk, LANE), jnp.float32),
          pltpu.VMEM_SHARED((pad_n, LANE), jnp.float32),
          pltpu.SemaphoreType.DMA,
          pltpu.SemaphoreType.DMA,
      ],
  )
  def deg_kernel(dst_hbm, out_hbm, dst_v, ones_v, zbuf, acc, sem_a, sem_b):
    c = lax.axis_index("c")
    s = lax.axis_index("s")
    w = c * NS + s

    one16 = jnp.full((LANE,), 1.0, jnp.float32)
    zero16 = jnp.zeros((LANE,), jnp.float32)
    for r in range(ROW):
      ones_v[r, pl.ds(0, LANE)] = one16
    for r in range(zchunk):
      zbuf[r, pl.ds(0, LANE)] = zero16
    base = s * rows_per_tile
    for t in range(rows_per_tile // zchunk):
      pltpu.sync_copy(zbuf, acc.at[pl.ds(base + t * zchunk, zchunk)])
    plsc.subcore_barrier()

    pltpu.sync_copy(dst_hbm.at[pl.ds(w * rows_per_w, rows_per_w)], dst_v)

    sems = [sem_a, sem_b]
    descs = [None, None]
    for j in range(rows_per_w):
      b = j % 2
      if descs[b] is not None:
        descs[b].wait()
      descs[b] = pltpu.async_copy(ones_v, acc.at[dst_v.at[j]], sems[b],
                                  add=True)
    for b in range(2):
      if descs[b] is not None:
        descs[b].wait()
    plsc.subcore_barrier()

    pltpu.sync_copy(acc.at[pl.ds(base, rows_per_tile)],
                    out_hbm.at[c, pl.ds(base, rows_per_tile)])

  return deg_kernel


def _make_prop_kernel(pad_n, rows_per_w, hdim):
  rows_per_tile = pad_n // NS
  zchunk = 64
  mesh = plsc.VectorSubcoreMesh(core_axis_name="c", subcore_axis_name="s")

  @functools.partial(
      pl.kernel,
      out_type=jax.ShapeDtypeStruct((NC, pad_n, hdim), jnp.float32),
      mesh=mesh,
      scratch_types=[
          pltpu.VMEM((rows_per_w, ROW), jnp.int32),
          pltpu.VMEM((rows_per_w, ROW), jnp.int32),
          pltpu.VMEM((2, ROW, hdim), jnp.float32),
          pltpu.VMEM((zchunk, hdim), jnp.float32),
          pltpu.VMEM_SHARED((pad_n, hdim), jnp.float32),
          pltpu.SemaphoreType.DMA,
          pltpu.SemaphoreType.DMA,
          pltpu.SemaphoreType.DMA,
          pltpu.SemaphoreType.DMA,
      ],
  )
  def prop_kernel(g_hbm, src_hbm, dst_hbm, out_hbm, src_v, dst_v, buf, zbuf,
                  acc, gsem_a, gsem_b, ssem_a, ssem_b):
    c = lax.axis_index("c")
    s = lax.axis_index("s")
    w = c * NS + s

    zero16 = jnp.zeros((LANE,), jnp.float32)
    for r in range(zchunk):
      for q in range(hdim // LANE):
        zbuf[r, pl.ds(q * LANE, LANE)] = zero16
    base = s * rows_per_tile
    for t in range(rows_per_tile // zchunk):
      pltpu.sync_copy(zbuf, acc.at[pl.ds(base + t * zchunk, zchunk)])
    plsc.subcore_barrier()

    pltpu.sync_copy(src_hbm.at[pl.ds(w * rows_per_w, rows_per_w)], src_v)
    pltpu.sync_copy(dst_hbm.at[pl.ds(w * rows_per_w, rows_per_w)], dst_v)

    gsems = [gsem_a, gsem_b]
    ssems = [ssem_a, ssem_b]
    g_descs = [None, None]
    s_descs = [None, None]
    g_descs[0] = pltpu.async_copy(g_hbm.at[src_v.at[0]], buf.at[0], gsems[0])
    for j in range(rows_per_w):
      b = j % 2
      nb = (j + 1) % 2
      if j + 1 < rows_per_w:
        if s_descs[nb] is not None:
          s_descs[nb].wait()
          s_descs[nb] = None
        g_descs[nb] = pltpu.async_copy(g_hbm.at[src_v.at[j + 1]], buf.at[nb],
                                       gsems[nb])
      g_descs[b].wait()
      s_descs[b] = pltpu.async_copy(buf.at[b], acc.at[dst_v.at[j]], ssems[b],
                                    add=True)
    for b in range(2):
      if s_descs[b] is not None:
        s_descs[b].wait()
    plsc.subcore_barrier()

    pltpu.sync_copy(acc.at[pl.ds(base, rows_per_tile)],
                    out_hbm.at[c, pl.ds(base, rows_per_tile)])

  return prop_kernel


def _tc_a_body(n, x_ref, w1_ref, degp_ref, g1_ref, dinv_ref):
  deg = degp_ref[0, 0:n, 0:1] + degp_ref[1, 0:n, 0:1]
  dinv = lax.rsqrt(jnp.maximum(deg, 1.0))
  h = jnp.dot(x_ref[...], w1_ref[...], preferred_element_type=jnp.float32)
  g1_ref[...] = h * dinv
  dinv_ref[...] = dinv


def _tc_b_body(n, p_ref, g1_ref, dinv_ref, b1_ref, g2_ref):
  s = p_ref[0, 0:n, :] + p_ref[1, 0:n, :] + g1_ref[...]
  h1 = jnp.maximum(dinv_ref[...] * s + b1_ref[...], 0.0)
  g2_ref[...] = dinv_ref[...] * h1


def _tc_c_body(n, q_ref, g2_ref, dinv_ref, w2_ref, b2_ref, wf1_ref, bf1_ref,
               wf2_ref, bf2_ref, out_ref):
  s = q_ref[0, 0:n, :] + q_ref[1, 0:n, :] + g2_ref[...]
  a = dinv_ref[...] * s
  h2 = jnp.dot(a, w2_ref[...], preferred_element_type=jnp.float32)
  h2 = h2 + b2_ref[...]
  f = jnp.dot(h2, wf1_ref[...], preferred_element_type=jnp.float32)
  f = f + bf1_ref[...]
  f = jnp.where(f > 0.0, f, jnp.expm1(f))
  lo = jnp.dot(f, wf2_ref[...], preferred_element_type=jnp.float32)
  lo = lo + bf2_ref[...]
  m = jnp.max(lo, axis=1, keepdims=True)
  z = lo - m
  out_ref[...] = z - jnp.log(jnp.sum(jnp.exp(z), axis=1, keepdims=True))


def kernel(x, edge_index, W1, b1, W2, b2, Wf1, bf1, Wf2, bf2):
  n, d = x.shape
  e = edge_index.shape[1]
  h1 = W1.shape[1]
  c_out = Wf2.shape[1]

  pad_n = _cdiv(n + 1, NS * 64) * NS * 64
  tot_rows = _cdiv(e, ROW)
  tot_rows = _cdiv(tot_rows, NW) * NW
  rows_per_w = tot_rows // NW
  e_pad = tot_rows * ROW - e

  src = jnp.concatenate(
      [edge_index[0], jnp.zeros((e_pad,), jnp.int32)]).reshape(tot_rows, ROW)
  dst = jnp.concatenate(
      [edge_index[1], jnp.full((e_pad,), n, jnp.int32)]).reshape(tot_rows, ROW)

  deg_k = _make_deg_kernel(pad_n, rows_per_w)
  prop_k = _make_prop_kernel(pad_n, rows_per_w, h1)

  degp = deg_k(dst)

  g1, dinv = pl.pallas_call(
      functools.partial(_tc_a_body, n),
      out_shape=(
          jax.ShapeDtypeStruct((n, h1), jnp.float32),
          jax.ShapeDtypeStruct((n, 1), jnp.float32),
      ),
  )(x, W1, degp)

  p = prop_k(g1, src, dst)

  g2 = pl.pallas_call(
      functools.partial(_tc_b_body, n),
      out_shape=jax.ShapeDtypeStruct((n, h1), jnp.float32),
  )(p, g1, dinv, b1.reshape(1, h1))

  q = prop_k(g2, src, dst)

  out = pl.pallas_call(
      functools.partial(_tc_c_body, n),
      out_shape=jax.ShapeDtypeStruct((n, c_out), jnp.float32),
  )(q, g2, dinv, W2, b2.reshape(1, -1), Wf1, bf1.reshape(1, -1), Wf2,
    bf2.reshape(1, -1))
  return out


# speedup vs baseline: 27.9032x; 27.9032x over previous
import functools

import jax
import jax.numpy as jnp
from jax import lax
from jax.experimental import pallas as pl
from jax.experimental.pallas import tpu as pltpu
from jax.experimental.pallas import tpu_sc as plsc

NC = 2
NS = 16
NW = NC * NS
LANE = 16
ROW = 128

_SC_PARAMS = pltpu.CompilerParams(use_tc_tiling_on_sc=False)


def _cdiv(a, b):
  return (a + b - 1) // b


def _mesh():
  return plsc.VectorSubcoreMesh(core_axis_name="c", subcore_axis_name="s")


def _zero_rows(zbuf, zchunk, hdim):
  zero16 = jnp.zeros((LANE,), jnp.float32)
  for r in range(zchunk):
    for q in range(hdim // LANE):
      zbuf[r, pl.ds(q * LANE, LANE)] = zero16


def _make_deg_kernel(pad_n, rows_per_w):
  rows_per_tile = NC * pad_n // NW
  zchunk = 64

  @functools.partial(
      pl.kernel,
      out_type=jax.ShapeDtypeStruct((NC * pad_n, LANE), jnp.float32),
      mesh=_mesh(),
      compiler_params=_SC_PARAMS,
      scratch_types=[
          pltpu.VMEM((ROW,), jnp.int32),
          pltpu.VMEM((ROW,), jnp.int32),
          pltpu.VMEM((ROW, LANE), jnp.float32),
          pltpu.VMEM((zchunk, LANE), jnp.float32),
          pltpu.VMEM_SHARED((NC * pad_n, LANE), jnp.float32),
          pltpu.SemaphoreType.DMA,
          pltpu.SemaphoreType.DMA,
          pltpu.SemaphoreType.DMA,
      ],
  )
  def deg_kernel(dst_hbm, out_hbm, idx_a, idx_b, ones_v, zbuf, acc,
                 isem, ssem_a, ssem_b):
    c = lax.axis_index("c")
    s = lax.axis_index("s")
    w = c * NS + s
    ebase = w * rows_per_w * ROW
    base = w * rows_per_tile

    one16 = jnp.full((LANE,), 1.0, jnp.float32)
    for r in range(ROW):
      ones_v[r, pl.ds(0, LANE)] = one16
    _zero_rows(zbuf, zchunk, LANE)
    for t in range(rows_per_tile // zchunk):
      pltpu.sync_copy(zbuf, acc.at[pl.ds(base + t * zchunk, zchunk)])
    plsc.subcore_barrier()

    idx = [idx_a, idx_b]
    ssems = [ssem_a, ssem_b]
    i_descs = [None, None]
    s_descs = [None, None]
    i_descs[0] = pltpu.async_copy(dst_hbm.at[pl.ds(ebase, ROW)], idx_a, isem)
    for j in range(rows_per_w):
      b = j % 2
      nb = (j + 1) % 2
      i_descs[b].wait()
      if j + 1 < rows_per_w:
        if s_descs[nb] is not None:
          s_descs[nb].wait()
          s_descs[nb] = None
        i_descs[nb] = pltpu.async_copy(
            dst_hbm.at[pl.ds(ebase + (j + 1) * ROW, ROW)], idx[nb], isem)
      s_descs[b] = pltpu.async_copy(ones_v, acc.at[idx[b]], ssems[b],
                                    add=True)
    for b in range(2):
      if s_descs[b] is not None:
        s_descs[b].wait()
    plsc.subcore_barrier()

    pltpu.sync_copy(acc.at[pl.ds(base, rows_per_tile)],
                    out_hbm.at[pl.ds(base, rows_per_tile)])

  return deg_kernel


def _make_prop_kernel(pad_n, rows_per_w, hdim):
  rows_per_tile = NC * pad_n // NW
  zchunk = 64

  @functools.partial(
      pl.kernel,
      out_type=jax.ShapeDtypeStruct((NC * pad_n, hdim), jnp.float32),
      mesh=_mesh(),
      compiler_params=_SC_PARAMS,
      scratch_types=[
          pltpu.VMEM((ROW,), jnp.int32),
          pltpu.VMEM((ROW,), jnp.int32),
          pltpu.VMEM((ROW,), jnp.int32),
          pltpu.VMEM((ROW,), jnp.int32),
          pltpu.VMEM((2, ROW, hdim), jnp.float32),
          pltpu.VMEM((zchunk, hdim), jnp.float32),
          pltpu.VMEM_SHARED((NC * pad_n, hdim), jnp.float32),
          pltpu.SemaphoreType.DMA,
          pltpu.SemaphoreType.DMA,
          pltpu.SemaphoreType.DMA,
          pltpu.SemaphoreType.DMA,
          pltpu.SemaphoreType.DMA,
          pltpu.SemaphoreType.DMA,
      ],
  )
  def prop_kernel(g_hbm, src_hbm, dst_hbm, out_hbm, sidx_a, sidx_b, didx_a,
                  didx_b, buf, zbuf, acc, isem_a, isem_b, gsem_a, gsem_b,
                  ssem_a, ssem_b):
    c = lax.axis_index("c")
    s = lax.axis_index("s")
    w = c * NS + s
    ebase = w * rows_per_w * ROW
    base = w * rows_per_tile

    _zero_rows(zbuf, zchunk, hdim)
    for t in range(rows_per_tile // zchunk):
      pltpu.sync_copy(zbuf, acc.at[pl.ds(base + t * zchunk, zchunk)])
    plsc.subcore_barrier()

    sidx = [sidx_a, sidx_b]
    didx = [didx_a, didx_b]
    gsems = [gsem_a, gsem_b]
    ssems = [ssem_a, ssem_b]
    si_descs = [None, None]
    di_descs = [None, None]
    g_descs = [None, None]
    s_descs = [None, None]
    si_descs[0] = pltpu.async_copy(src_hbm.at[pl.ds(ebase, ROW)], sidx_a,
                                   isem_a)
    di_descs[0] = pltpu.async_copy(dst_hbm.at[pl.ds(ebase, ROW)], didx_a,
                                   isem_b)
    for j in range(rows_per_w):
      b = j % 2
      nb = (j + 1) % 2
      si_descs[b].wait()
      di_descs[b].wait()
      g_descs[b] = pltpu.async_copy(g_hbm.at[sidx[b]], buf.at[b], gsems[b])
      if j + 1 < rows_per_w:
        if s_descs[nb] is not None:
          s_descs[nb].wait()
          s_descs[nb] = None
        si_descs[nb] = pltpu.async_copy(
            src_hbm.at[pl.ds(ebase + (j + 1) * ROW, ROW)], sidx[nb], isem_a)
        di_descs[nb] = pltpu.async_copy(
            dst_hbm.at[pl.ds(ebase + (j + 1) * ROW, ROW)], didx[nb], isem_b)
      g_descs[b].wait()
      s_descs[b] = pltpu.async_copy(buf.at[b], acc.at[didx[b]], ssems[b],
                                    add=True)
    for b in range(2):
      if s_descs[b] is not None:
        s_descs[b].wait()
    plsc.subcore_barrier()

    pltpu.sync_copy(acc.at[pl.ds(base, rows_per_tile)],
                    out_hbm.at[pl.ds(base, rows_per_tile)])

  return prop_kernel


def _tc_a_body(n, pad_n, x_ref, w1_ref, degp_ref, g1_ref, dinv_ref):
  deg = degp_ref[0:n, 0:1] + degp_ref[pad_n:pad_n + n, 0:1]
  dinv = lax.rsqrt(jnp.maximum(deg, 1.0))
  h = jnp.dot(x_ref[...], w1_ref[...], preferred_element_type=jnp.float32)
  g1_ref[...] = h * dinv
  dinv_ref[...] = dinv


def _tc_b_body(n, pad_n, p_ref, g1_ref, dinv_ref, b1_ref, g2_ref):
  s = p_ref[0:n, :] + p_ref[pad_n:pad_n + n, :] + g1_ref[...]
  h1 = jnp.maximum(dinv_ref[...] * s + b1_ref[...], 0.0)
  g2_ref[...] = dinv_ref[...] * h1


def _tc_c_body(n, pad_n, q_ref, g2_ref, dinv_ref, w2_ref, b2_ref, wf1_ref,
               bf1_ref, wf2_ref, bf2_ref, out_ref):
  s = q_ref[0:n, :] + q_ref[pad_n:pad_n + n, :] + g2_ref[...]
  a = dinv_ref[...] * s
  h2 = jnp.dot(a, w2_ref[...], preferred_element_type=jnp.float32)
  h2 = h2 + b2_ref[...]
  f = jnp.dot(h2, wf1_ref[...], preferred_element_type=jnp.float32)
  f = f + bf1_ref[...]
  f = jnp.where(f > 0.0, f, jnp.exp(jnp.minimum(f, 0.0)) - 1.0)
  lo = jnp.dot(f, wf2_ref[...], preferred_element_type=jnp.float32)
  lo = lo + bf2_ref[...]
  m = jnp.max(lo, axis=1, keepdims=True)
  z = lo - m
  out_ref[...] = z - jnp.log(jnp.sum(jnp.exp(z), axis=1, keepdims=True))


def kernel(x, edge_index, W1, b1, W2, b2, Wf1, bf1, Wf2, bf2):
  n, d = x.shape
  e = edge_index.shape[1]
  h1 = W1.shape[1]
  c_out = Wf2.shape[1]

  pad_n = _cdiv(n + 1, NS * 64) * NS * 64
  rows_per_w = _cdiv(_cdiv(e, ROW), NW)
  tot_rows = rows_per_w * NW
  e_tot = tot_rows * ROW
  e_pad = e_tot - e

  src = jnp.concatenate([edge_index[0], jnp.zeros((e_pad,), jnp.int32)])
  dst = jnp.concatenate([edge_index[1], jnp.full((e_pad,), n, jnp.int32)])
  half = jnp.arange(e_tot, dtype=jnp.int32) >= (e_tot // 2)
  dst = dst + jnp.where(half, jnp.int32(pad_n), jnp.int32(0))

  deg_k = _make_deg_kernel(pad_n, rows_per_w)
  prop_k = _make_prop_kernel(pad_n, rows_per_w, h1)

  degp = deg_k(dst)

  g1, dinv = pl.pallas_call(
      functools.partial(_tc_a_body, n, pad_n),
      out_shape=(
          jax.ShapeDtypeStruct((n, h1), jnp.float32),
          jax.ShapeDtypeStruct((n, 1), jnp.float32),
      ),
  )(x, W1, degp)

  p = prop_k(g1, src, dst)

  g2 = pl.pallas_call(
      functools.partial(_tc_b_body, n, pad_n),
      out_shape=jax.ShapeDtypeStruct((n, h1), jnp.float32),
  )(p, g1, dinv, b1.reshape(1, h1))

  q = prop_k(g2, src, dst)

  out = pl.pallas_call(
      functools.partial(_tc_c_body, n, pad_n),
      out_shape=jax.ShapeDtypeStruct((n, c_out), jnp.float32),
  )(q, g2, dinv, W2, b2.reshape(1, -1), Wf1, bf1.reshape(1, -1), Wf2,
    bf2.reshape(1, -1))
  return out


# speedup vs baseline: 35.4928x; 1.2720x over previous
import functools

import jax
import jax.numpy as jnp
from jax import lax
from jax.experimental import pallas as pl
from jax.experimental.pallas import tpu as pltpu
from jax.experimental.pallas import tpu_sc as plsc

NC = 2
NS = 16
NW = NC * NS
LANE = 16
ROW = 128

_SC_PARAMS = pltpu.CompilerParams(use_tc_tiling_on_sc=False)


def _cdiv(a, b):
  return (a + b - 1) // b


def _mesh():
  return plsc.VectorSubcoreMesh(core_axis_name="c", subcore_axis_name="s")


def _zero_rows(zbuf, zchunk, hdim):
  zero16 = jnp.zeros((LANE,), jnp.float32)
  for r in range(zchunk):
    for q in range(hdim // LANE):
      zbuf[r, pl.ds(q * LANE, LANE)] = zero16


def _fill_idx(idx1d, staged, j):
  for k in range(ROW // LANE):
    idx1d[pl.ds(k * LANE, LANE)] = staged[pl.ds(j * ROW + k * LANE, LANE)]


def _make_deg_kernel(pad_n, rows_per_w):
  rows_per_tile = NC * pad_n // NW
  zchunk = 64
  nbuf = 4

  @functools.partial(
      pl.kernel,
      out_type=jax.ShapeDtypeStruct((NC * pad_n, LANE), jnp.float32),
      mesh=_mesh(),
      compiler_params=_SC_PARAMS,
      scratch_types=[
          pltpu.VMEM((rows_per_w * ROW,), jnp.int32),
          [pltpu.VMEM((ROW,), jnp.int32) for _ in range(nbuf)],
          pltpu.VMEM((ROW, LANE), jnp.float32),
          pltpu.VMEM((zchunk, LANE), jnp.float32),
          pltpu.VMEM_SHARED((NC * pad_n, LANE), jnp.float32),
          [pltpu.SemaphoreType.DMA for _ in range(nbuf)],
      ],
  )
  def deg_kernel(dst_hbm, out_hbm, staged, idx, ones_v, zbuf, acc, ssems):
    c = lax.axis_index("c")
    s = lax.axis_index("s")
    w = c * NS + s
    ebase = w * rows_per_w * ROW
    base = w * rows_per_tile

    one16 = jnp.full((LANE,), 1.0, jnp.float32)
    for r in range(ROW):
      ones_v[r, pl.ds(0, LANE)] = one16
    _zero_rows(zbuf, zchunk, LANE)
    for t in range(rows_per_tile // zchunk):
      pltpu.sync_copy(zbuf, acc.at[pl.ds(base + t * zchunk, zchunk)])
    pltpu.sync_copy(dst_hbm.at[pl.ds(ebase, rows_per_w * ROW)], staged)
    plsc.subcore_barrier()

    s_descs = [None] * nbuf
    for j in range(rows_per_w):
      b = j % nbuf
      if s_descs[b] is not None:
        s_descs[b].wait()
      _fill_idx(idx[b], staged, j)
      s_descs[b] = pltpu.async_copy(ones_v, acc.at[idx[b]], ssems[b],
                                    add=True)
    for b in range(nbuf):
      if s_descs[b] is not None:
        s_descs[b].wait()
    plsc.subcore_barrier()

    pltpu.sync_copy(acc.at[pl.ds(base, rows_per_tile)],
                    out_hbm.at[pl.ds(base, rows_per_tile)])

  return deg_kernel


def _make_prop_kernel(pad_n, rows_per_w, hdim):
  rows_per_tile = NC * pad_n // NW
  zchunk = 64

  nbuf = 4
  skew = 2

  @functools.partial(
      pl.kernel,
      out_type=jax.ShapeDtypeStruct((NC * pad_n, hdim), jnp.float32),
      mesh=_mesh(),
      compiler_params=_SC_PARAMS,
      scratch_types=[
          pltpu.VMEM((rows_per_w * ROW,), jnp.int32),
          pltpu.VMEM((rows_per_w * ROW,), jnp.int32),
          [pltpu.VMEM((ROW,), jnp.int32) for _ in range(nbuf)],
          [pltpu.VMEM((ROW,), jnp.int32) for _ in range(nbuf)],
          pltpu.VMEM((nbuf, ROW, hdim), jnp.float32),
          pltpu.VMEM((zchunk, hdim), jnp.float32),
          pltpu.VMEM_SHARED((NC * pad_n, hdim), jnp.float32),
          [pltpu.SemaphoreType.DMA for _ in range(nbuf)],
          [pltpu.SemaphoreType.DMA for _ in range(nbuf)],
      ],
  )
  def prop_kernel(g_hbm, src_hbm, dst_hbm, out_hbm, sstage, dstage, sidx,
                  didx, buf, zbuf, acc, gsems, ssems):
    c = lax.axis_index("c")
    s = lax.axis_index("s")
    w = c * NS + s
    ebase = w * rows_per_w * ROW
    base = w * rows_per_tile

    _zero_rows(zbuf, zchunk, hdim)
    for t in range(rows_per_tile // zchunk):
      pltpu.sync_copy(zbuf, acc.at[pl.ds(base + t * zchunk, zchunk)])
    pltpu.sync_copy(src_hbm.at[pl.ds(ebase, rows_per_w * ROW)], sstage)
    pltpu.sync_copy(dst_hbm.at[pl.ds(ebase, rows_per_w * ROW)], dstage)
    plsc.subcore_barrier()

    g_descs = [None] * nbuf
    s_descs = [None] * nbuf
    for j in range(rows_per_w + skew):
      if j < rows_per_w:
        b = j % nbuf
        if s_descs[b] is not None:
          s_descs[b].wait()
          s_descs[b] = None
        _fill_idx(sidx[b], sstage, j)
        _fill_idx(didx[b], dstage, j)
        g_descs[b] = pltpu.async_copy(g_hbm.at[sidx[b]], buf.at[b], gsems[b])
      if j >= skew:
        k = j - skew
        kb = k % nbuf
        g_descs[kb].wait()
        s_descs[kb] = pltpu.async_copy(buf.at[kb], acc.at[didx[kb]],
                                       ssems[kb], add=True)
    for b in range(nbuf):
      if s_descs[b] is not None:
        s_descs[b].wait()
    plsc.subcore_barrier()

    pltpu.sync_copy(acc.at[pl.ds(base, rows_per_tile)],
                    out_hbm.at[pl.ds(base, rows_per_tile)])

  return prop_kernel


def _tc_a_body(n, pad_n, x_ref, w1_ref, degp_ref, g1_ref, dinv_ref):
  deg = degp_ref[0:n, 0:1] + degp_ref[pad_n:pad_n + n, 0:1]
  dinv = lax.rsqrt(jnp.maximum(deg, 1.0))
  h = jnp.dot(x_ref[...], w1_ref[...], preferred_element_type=jnp.float32)
  g1_ref[...] = h * dinv
  dinv_ref[...] = dinv


def _tc_b_body(n, pad_n, p_ref, g1_ref, dinv_ref, b1_ref, g2_ref):
  s = p_ref[0:n, :] + p_ref[pad_n:pad_n + n, :] + g1_ref[...]
  h1 = jnp.maximum(dinv_ref[...] * s + b1_ref[...], 0.0)
  g2_ref[...] = dinv_ref[...] * h1


def _tc_c_body(n, pad_n, q_ref, g2_ref, dinv_ref, w2_ref, b2_ref, wf1_ref,
               bf1_ref, wf2_ref, bf2_ref, out_ref):
  s = q_ref[0:n, :] + q_ref[pad_n:pad_n + n, :] + g2_ref[...]
  a = dinv_ref[...] * s
  h2 = jnp.dot(a, w2_ref[...], preferred_element_type=jnp.float32)
  h2 = h2 + b2_ref[...]
  f = jnp.dot(h2, wf1_ref[...], preferred_element_type=jnp.float32)
  f = f + bf1_ref[...]
  f = jnp.where(f > 0.0, f, jnp.exp(jnp.minimum(f, 0.0)) - 1.0)
  lo = jnp.dot(f, wf2_ref[...], preferred_element_type=jnp.float32)
  lo = lo + bf2_ref[...]
  m = jnp.max(lo, axis=1, keepdims=True)
  z = lo - m
  out_ref[...] = z - jnp.log(jnp.sum(jnp.exp(z), axis=1, keepdims=True))


def kernel(x, edge_index, W1, b1, W2, b2, Wf1, bf1, Wf2, bf2):
  n, d = x.shape
  e = edge_index.shape[1]
  h1 = W1.shape[1]
  c_out = Wf2.shape[1]

  pad_n = _cdiv(n + 1, NS * 64) * NS * 64
  rows_per_w = _cdiv(_cdiv(e, ROW), NW)
  tot_rows = rows_per_w * NW
  e_tot = tot_rows * ROW
  e_pad = e_tot - e

  src = jnp.concatenate([edge_index[0], jnp.zeros((e_pad,), jnp.int32)])
  dst = jnp.concatenate([edge_index[1], jnp.full((e_pad,), n, jnp.int32)])
  half = jnp.arange(e_tot, dtype=jnp.int32) >= (e_tot // 2)
  dst = dst + jnp.where(half, jnp.int32(pad_n), jnp.int32(0))

  deg_k = _make_deg_kernel(pad_n, rows_per_w)
  prop_k = _make_prop_kernel(pad_n, rows_per_w, h1)

  degp = deg_k(dst)

  g1, dinv = pl.pallas_call(
      functools.partial(_tc_a_body, n, pad_n),
      out_shape=(
          jax.ShapeDtypeStruct((n, h1), jnp.float32),
          jax.ShapeDtypeStruct((n, 1), jnp.float32),
      ),
  )(x, W1, degp)

  p = prop_k(g1, src, dst)

  g2 = pl.pallas_call(
      functools.partial(_tc_b_body, n, pad_n),
      out_shape=jax.ShapeDtypeStruct((n, h1), jnp.float32),
  )(p, g1, dinv, b1.reshape(1, h1))

  q = prop_k(g2, src, dst)

  out = pl.pallas_call(
      functools.partial(_tc_c_body, n, pad_n),
      out_shape=jax.ShapeDtypeStruct((n, c_out), jnp.float32),
  )(q, g2, dinv, W2, b2.reshape(1, -1), Wf1, bf1.reshape(1, -1), Wf2,
    bf2.reshape(1, -1))
  return out
